# baseline (speedup 1.0000x reference)
import jax
import jax.numpy as jnp
import numpy as np
from jax import lax
from jax.experimental import pallas as pl
from jax.experimental.pallas import tpu as pltpu

_TINY = np.float32(1.1754943508222875e-38)
_SB = 8
_K0 = 0
_K1 = 42
_K2 = _K0 ^ _K1 ^ 0x1BD11BDA


def _rotl(x, d):
    return lax.shift_left(x, jnp.int32(d)) | lax.shift_right_logical(
        x, jnp.int32(32 - d))


def _threefry_bits(i_plus_k1):
    ks = [jnp.int32(_K0), jnp.int32(_K1), jnp.int32(_K2)]
    rot0 = (13, 15, 26, 6)
    rot1 = (17, 29, 16, 24)
    x0 = jnp.zeros_like(i_plus_k1) + ks[0]
    x1 = i_plus_k1

    def rounds(x0, x1, rots):
        for r in rots:
            x0 = x0 + x1
            x1 = x0 ^ _rotl(x1, r)
        return x0, x1

    x0, x1 = rounds(x0, x1, rot0)
    x0 = x0 + ks[1]
    x1 = x1 + (ks[2] + jnp.int32(1))
    x0, x1 = rounds(x0, x1, rot1)
    x0 = x0 + ks[2]
    x1 = x1 + (ks[0] + jnp.int32(2))
    x0, x1 = rounds(x0, x1, rot0)
    x0 = x0 + ks[0]
    x1 = x1 + (ks[1] + jnp.int32(3))
    x0, x1 = rounds(x0, x1, rot1)
    x0 = x0 + ks[1]
    x1 = x1 + (ks[2] + jnp.int32(4))
    x0, x1 = rounds(x0, x1, rot0)
    x0 = x0 + ks[2]
    x1 = x1 + (ks[0] + jnp.int32(5))
    return x0 ^ x1


def _gumbel_from_bits(bits):
    fbits = lax.shift_right_logical(bits, jnp.int32(9)) | jnp.int32(0x3F800000)
    f = lax.bitcast_convert_type(fbits, jnp.float32) - np.float32(1.0)
    u = f + _TINY
    return -jnp.log(-jnp.log(u))


def _stats_kernel(z_ref, mu_ref, inv_ref):
    blk = z_ref[...]
    b3 = blk.reshape(128, 8, 2048)
    s = jnp.sum(b3, axis=2)
    q = jnp.sum(b3 * b3, axis=2)
    s_t = s.T
    q_t = q.T
    inv_n = jnp.float32(1.0 / 2048.0)
    mu = s_t * inv_n
    var = q_t * inv_n - mu * mu
    den = jnp.sqrt(jnp.maximum(var, 0.0)) + jnp.float32(0.001)
    mu_ref[0] = mu
    inv_ref[0] = jnp.float32(1.0) / den


def _sample_kernel(z_ref, mu_ref, inv_ref, out_ref):
    b0 = pl.program_id(0) * _SB
    a_i = lax.broadcasted_iota(jnp.int32, (_SB, 8, 128), 0)
    s_i = lax.broadcasted_iota(jnp.int32, (_SB, 8, 128), 1)
    l_i = lax.broadcasted_iota(jnp.int32, (_SB, 8, 128), 2)
    ibase = (b0 + a_i) * jnp.int32(16384) + l_i * jnp.int32(8) + s_i + jnp.int32(_K1)
    jbase = l_i * jnp.int32(8) + s_i
    cols = []
    for hq in range(8):
        m = jnp.full((_SB, 8, 128), -jnp.inf, jnp.float32)
        jw = jnp.zeros((_SB, 8, 128), jnp.int32)
        for cc in range(2):
            c = hq * 2 + cc
            v3 = z_ref[:, c * 128:(c + 1) * 128].reshape(_SB, 8, 128)
            mu_t = mu_ref[c].reshape(1, 8, 128)
            inv_t = inv_ref[c].reshape(1, 8, 128)
            std = (v3 - mu_t) * inv_t
            lin = ibase + jnp.int32(2048 * hq + 1024 * cc)
            g = _gumbel_from_bits(_threefry_bits(lin))
            val = std + g
            upd = val > m
            m = jnp.where(upd, val, m)
            jw = jnp.where(upd, jbase + jnp.int32(1024 * cc), jw)
        mx = jnp.max(jnp.max(m, axis=2, keepdims=True), axis=1, keepdims=True)
        cand = jnp.where(m == mx, jw, jnp.int32(0x7FFFFFFF))
        cols.append(jnp.min(jnp.min(cand, axis=2), axis=1, keepdims=True))
    out_ref[...] = jnp.concatenate(cols, axis=1)


@jax.jit
def kernel(x):
    B, H, W = x.shape
    z = x.transpose(0, 2, 1).reshape(B * W, H)
    mu_t, inv_t = pl.pallas_call(
        _stats_kernel,
        grid=(16,),
        in_specs=[pl.BlockSpec((1024, H), lambda k: (k, 0))],
        out_specs=[
            pl.BlockSpec((1, 8, 128), lambda k: (k, 0, 0)),
            pl.BlockSpec((1, 8, 128), lambda k: (k, 0, 0)),
        ],
        out_shape=[
            jax.ShapeDtypeStruct((16, 8, 128), jnp.float32),
            jax.ShapeDtypeStruct((16, 8, 128), jnp.float32),
        ],
        compiler_params=pltpu.CompilerParams(
            dimension_semantics=("parallel",)),
    )(z)
    out = pl.pallas_call(
        _sample_kernel,
        grid=(B // _SB,),
        in_specs=[
            pl.BlockSpec((8 * _SB, H), lambda k: (k, 0)),
            pl.BlockSpec((16, 8, 128), lambda k: (0, 0, 0)),
            pl.BlockSpec((16, 8, 128), lambda k: (0, 0, 0)),
        ],
        out_specs=pl.BlockSpec((_SB, 8), lambda k: (k, 0)),
        out_shape=jax.ShapeDtypeStruct((B, W), jnp.int32),
        compiler_params=pltpu.CompilerParams(
            dimension_semantics=("parallel",)),
    )(z, mu_t, inv_t)
    return out.reshape(1, H, W)


# speedup vs baseline: 1.0597x; 1.0597x over previous
import jax
import jax.numpy as jnp
import numpy as np
from jax import lax
from jax.experimental import pallas as pl
from jax.experimental.pallas import tpu as pltpu

_TINY = np.float32(1.1754943508222875e-38)
_SB = 16
_K0 = 0
_K1 = 42
_K2 = _K0 ^ _K1 ^ 0x1BD11BDA


def _rotl(x, d):
    return lax.shift_left(x, jnp.int32(d)) | lax.shift_right_logical(
        x, jnp.int32(32 - d))


def _threefry_bits(i_plus_k1):
    ks = [jnp.int32(_K0), jnp.int32(_K1), jnp.int32(_K2)]
    rot0 = (13, 15, 26, 6)
    rot1 = (17, 29, 16, 24)
    x0 = jnp.zeros_like(i_plus_k1) + ks[0]
    x1 = i_plus_k1

    def rounds(x0, x1, rots):
        for r in rots:
            x0 = x0 + x1
            x1 = x0 ^ _rotl(x1, r)
        return x0, x1

    x0, x1 = rounds(x0, x1, rot0)
    x0 = x0 + ks[1]
    x1 = x1 + (ks[2] + jnp.int32(1))
    x0, x1 = rounds(x0, x1, rot1)
    x0 = x0 + ks[2]
    x1 = x1 + (ks[0] + jnp.int32(2))
    x0, x1 = rounds(x0, x1, rot0)
    x0 = x0 + ks[0]
    x1 = x1 + (ks[1] + jnp.int32(3))
    x0, x1 = rounds(x0, x1, rot1)
    x0 = x0 + ks[1]
    x1 = x1 + (ks[2] + jnp.int32(4))
    x0, x1 = rounds(x0, x1, rot0)
    x0 = x0 + ks[2]
    x1 = x1 + (ks[0] + jnp.int32(5))
    return x0 ^ x1


def _gumbel_from_bits(bits):
    fbits = lax.shift_right_logical(bits, jnp.int32(9)) | jnp.int32(0x3F800000)
    f = lax.bitcast_convert_type(fbits, jnp.float32) - np.float32(1.0)
    u = f + _TINY
    return -jnp.log(-jnp.log(u))


def _stats_kernel(z_ref, mu_ref, inv_ref):
    blk = z_ref[...]
    b3 = blk.reshape(128, 8, 2048)
    s = jnp.sum(b3, axis=2)
    q = jnp.sum(b3 * b3, axis=2)
    s_t = s.T
    q_t = q.T
    inv_n = jnp.float32(1.0 / 2048.0)
    mu = s_t * inv_n
    var = q_t * inv_n - mu * mu
    den = jnp.sqrt(jnp.maximum(var, 0.0)) + jnp.float32(0.001)
    mu_ref[0] = mu
    inv_ref[0] = jnp.float32(1.0) / den


def _sample_kernel(z_ref, mu_ref, inv_ref, out_ref):
    b0 = pl.program_id(0) * _SB
    a_i = lax.broadcasted_iota(jnp.int32, (_SB, 8, 128), 0)
    s_i = lax.broadcasted_iota(jnp.int32, (_SB, 8, 128), 1)
    l_i = lax.broadcasted_iota(jnp.int32, (_SB, 8, 128), 2)
    ibase = (b0 + a_i) * jnp.int32(16384) + l_i * jnp.int32(8) + s_i + jnp.int32(_K1)
    jbase = l_i * jnp.int32(8) + s_i
    cols = []
    for hq in range(8):
        vals = []
        for cc in range(2):
            c = hq * 2 + cc
            v3 = z_ref[:, c * 128:(c + 1) * 128].reshape(_SB, 8, 128)
            mu_t = mu_ref[c].reshape(1, 8, 128)
            inv_t = inv_ref[c].reshape(1, 8, 128)
            std = (v3 - mu_t) * inv_t
            lin = ibase + jnp.int32(2048 * hq + 1024 * cc)
            vals.append(std + _gumbel_from_bits(_threefry_bits(lin)))
        m = jnp.maximum(vals[0], vals[1])
        mx = jnp.max(jnp.max(m, axis=2, keepdims=True), axis=1, keepdims=True)
        big = jnp.int32(0x7FFFFFFF)
        cand = jnp.minimum(
            jnp.where(vals[0] == mx, jbase, big),
            jnp.where(vals[1] == mx, jbase + jnp.int32(1024), big))
        cols.append(jnp.min(jnp.min(cand, axis=2), axis=1, keepdims=True))
    out_ref[...] = jnp.concatenate(cols, axis=1)


@jax.jit
def kernel(x):
    B, H, W = x.shape
    z = x.transpose(0, 2, 1).reshape(B * W, H)
    mu_t, inv_t = pl.pallas_call(
        _stats_kernel,
        grid=(16,),
        in_specs=[pl.BlockSpec((1024, H), lambda k: (k, 0))],
        out_specs=[
            pl.BlockSpec((1, 8, 128), lambda k: (k, 0, 0)),
            pl.BlockSpec((1, 8, 128), lambda k: (k, 0, 0)),
        ],
        out_shape=[
            jax.ShapeDtypeStruct((16, 8, 128), jnp.float32),
            jax.ShapeDtypeStruct((16, 8, 128), jnp.float32),
        ],
        compiler_params=pltpu.CompilerParams(
            dimension_semantics=("parallel",)),
    )(z)
    out = pl.pallas_call(
        _sample_kernel,
        grid=(B // _SB,),
        in_specs=[
            pl.BlockSpec((8 * _SB, H), lambda k: (k, 0)),
            pl.BlockSpec((16, 8, 128), lambda k: (0, 0, 0)),
            pl.BlockSpec((16, 8, 128), lambda k: (0, 0, 0)),
        ],
        out_specs=pl.BlockSpec((_SB, 8), lambda k: (k, 0)),
        out_shape=jax.ShapeDtypeStruct((B, W), jnp.int32),
        compiler_params=pltpu.CompilerParams(
            dimension_semantics=("parallel",)),
    )(z, mu_t, inv_t)
    return out.reshape(1, H, W)
